# Initial kernel scaffold; baseline (speedup 1.0000x reference)
import jax
import jax.numpy as jnp
from jax.experimental import pallas as pl


def kernel(xyz, points, prev_feat, W1, g1, b1, Wv1, W2, g2, b2, Wv2):
    raise NotImplementedError("write your pallas kernel here")


# speedup vs baseline: 27.5742x; 27.5742x over previous
import functools

import jax
import jax.numpy as jnp
from jax.experimental import pallas as pl
from jax.experimental.pallas import tpu as pltpu

B = 4
N = 2048
G = 512
S = 32
T = G * S
TILE = 512
NT = T // TILE
GRID = B * NT
NTOT = float(B * T)
def _doth(a, b):
    return jax.lax.dot_general(a, b, (((1,), (0,)), ((), ())),
                               precision=jax.lax.Precision.HIGHEST,
                               preferred_element_type=jnp.float32)


def _dot(a, b):
    return jax.lax.dot_general(a, b, (((1,), (0,)), ((), ())),
                               precision=jax.lax.Precision.DEFAULT,
                               preferred_element_type=jnp.float32)


def _fps_body(xc_ref, xr_ref, idx_ref, q_ref, dist_ref):
    dist_ref[...] = jnp.full((B, N), 1e10, dtype=jnp.float32)
    for b in range(B):
        idx_ref[b, 0:1, 0:1] = jnp.zeros((1, 1), dtype=jnp.int32)

    lane = jax.lax.broadcasted_iota(jnp.int32, (1, N), 1)

    def body(i, last):
        new_last = []
        for b in range(B):
            lb = last[b]
            p = xr_ref[b, pl.ds(lb, 1), :]
            q_ref[b, pl.ds(i - 1, 1), :] = p
            px, py, pz = p[:, 0:1], p[:, 1:2], p[:, 2:3]
            dx = xc_ref[b, 0:1, :] - px
            dy = xc_ref[b, 1:2, :] - py
            dz = xc_ref[b, 2:3, :] - pz
            d = dx * dx + dy * dy + dz * dz
            nd = jnp.minimum(dist_ref[pl.ds(b, 1), :], d)
            dist_ref[pl.ds(b, 1), :] = nd
            m = jnp.max(nd)
            cand = jnp.where(nd == m, lane, N)
            nxt = jnp.min(cand)
            idx_ref[b, pl.ds(i, 1), 0:1] = jnp.reshape(nxt, (1, 1))
            new_last.append(nxt)
        return tuple(new_last)

    last = jax.lax.fori_loop(1, G, body, tuple(jnp.int32(0) for _ in range(B)))
    for b in range(B):
        p = xr_ref[b, pl.ds(last[b], 1), :]
        q_ref[b, pl.ds(G - 1, 1), :] = p


def _run_fps(xyz, xyz_rows):
    return pl.pallas_call(
        _fps_body,
        out_shape=(jax.ShapeDtypeStruct((B, G, 1), jnp.int32),
                   jax.ShapeDtypeStruct((B, G, 3), jnp.float32)),
        scratch_shapes=[pltpu.VMEM((B, N), jnp.float32)],
    )(xyz, xyz_rows)


def _knn_body(q_ref, xc_ref, idx_ref, d_ref):
    q = q_ref[0]
    qq, pp, qp = None, None, None
    for c in range(3):
        qc = q[:, c:c + 1]
        pc = xc_ref[0, c:c + 1, :]
        qb = qc.astype(jnp.bfloat16).astype(jnp.float32)
        pb = pc.astype(jnp.bfloat16).astype(jnp.float32)
        qq = qc * qc if qq is None else qq + qc * qc
        pp = pc * pc if pp is None else pp + pc * pc
        qp = qb * pb if qp is None else qp + qb * pb
    d_ref[...] = qq - 2.0 * qp + pp
    lane = jax.lax.broadcasted_iota(jnp.int32, (G, N), 1)
    picks = []
    for j in range(S):
        d = d_ref[...]
        m = jnp.min(d, axis=1, keepdims=True)
        cand = jnp.where(d == m, lane, N)
        amin = jnp.min(cand, axis=1, keepdims=True)
        picks.append(amin)
        d_ref[...] = jnp.where(lane == amin, 1e30, d)
    idx_ref[0] = jnp.concatenate(picks, axis=1)


def _run_knn(new_q, xyz):
    return pl.pallas_call(
        _knn_body,
        grid=(B,),
        in_specs=[pl.BlockSpec((1, G, 3), lambda b: (b, 0, 0)),
                  pl.BlockSpec((1, 3, N), lambda b: (b, 0, 0))],
        out_specs=pl.BlockSpec((1, G, S), lambda b: (b, 0, 0)),
        out_shape=jax.ShapeDtypeStruct((B, G, S), jnp.int32),
        scratch_shapes=[pltpu.VMEM((G, N), jnp.float32)],
    )(new_q, xyz)


def _passa_body(tbl_ref, idx_ref, qt_ref, w1t_ref, z1_ref):
    idxv = idx_ref[0]
    lane = jax.lax.broadcasted_iota(jnp.int32, (TILE, N), 1)
    onehot = (idxv == lane).astype(jnp.float32)
    CIN = tbl_ref.shape[-1]
    col0 = (jax.lax.broadcasted_iota(jnp.int32, (1, CIN), 1) == 0
            ).astype(jnp.float32)
    w1t = w1t_ref[...]
    zs = []
    for d in range(3):
        x = _doth(onehot, tbl_ref[0, d])
        x = x - qt_ref[0, :, d:d + 1] * col0
        z1_ref[0, d] = _dot(x, w1t)


def _run_passa(tbl, idxt, qt, w1t):
    CIN, C = w1t.shape
    return pl.pallas_call(
        _passa_body,
        grid=(GRID,),
        in_specs=[pl.BlockSpec((1, 3, N, CIN), lambda t: (t // NT, 0, 0, 0)),
                  pl.BlockSpec((1, TILE, 1), lambda t: (t, 0, 0)),
                  pl.BlockSpec((1, TILE, 4), lambda t: (t, 0, 0)),
                  pl.BlockSpec((CIN, C), lambda t: (0, 0))],
        out_specs=pl.BlockSpec((1, 3, TILE, C), lambda t: (t, 0, 0, 0)),
        out_shape=jax.ShapeDtypeStruct((GRID, 3, TILE, C), jnp.float32),
    )(tbl, idxt, qt, w1t)


def _vn_block(xs, wvt):
    vs = [_dot(x, wvt) for x in xs]
    vn = vs[0] * vs[0] + vs[1] * vs[1] + vs[2] * vs[2]
    sq = jnp.sqrt(vn + 1e-8)
    vs = [v / sq for v in vs]
    dt = xs[0] * vs[0] + xs[1] * vs[1] + xs[2] * vs[2]
    mask = (dt >= 0).astype(jnp.float32)
    return [mask * x + (1.0 - mask) * 0.2 * (x - dt * v)
            for x, v in zip(xs, vs)]


def _bn_factor(zs, st_ref, gb_ref):
    mean = st_ref[0:1, :]
    var = st_ref[1:2, :]
    nrm = jnp.sqrt(zs[0] * zs[0] + zs[1] * zs[1] + zs[2] * zs[2] + 1e-8)
    nbn = ((nrm - mean) / jnp.sqrt(var + 1e-5)) * gb_ref[0:1, :] + gb_ref[1:2, :]
    return nbn / nrm


def _passb_body(z1_ref, st1_ref, gb1_ref, wv1t_ref, w2t_ref, z2_ref):
    zs = [z1_ref[0, d] for d in range(3)]
    factor = _bn_factor(zs, st1_ref, gb1_ref)
    xs = [z * factor for z in zs]
    hs = _vn_block(xs, wv1t_ref[...])
    w2t = w2t_ref[...]
    for d in range(3):
        z2_ref[0, d] = _dot(hs[d], w2t)


def _run_passb(z1, st1, gb1, wv1t, w2t):
    C1, C2 = wv1t.shape[0], w2t.shape[1]
    return pl.pallas_call(
        _passb_body,
        grid=(GRID,),
        in_specs=[pl.BlockSpec((1, 3, TILE, C1), lambda t: (t, 0, 0, 0)),
                  pl.BlockSpec((8, C1), lambda t: (0, 0)),
                  pl.BlockSpec((2, C1), lambda t: (0, 0)),
                  pl.BlockSpec((C1, C1), lambda t: (0, 0)),
                  pl.BlockSpec((C1, C2), lambda t: (0, 0))],
        out_specs=pl.BlockSpec((1, 3, TILE, C2), lambda t: (t, 0, 0, 0)),
        out_shape=jax.ShapeDtypeStruct((GRID, 3, TILE, C2), jnp.float32),
    )(z1, st1, gb1, wv1t, w2t)


def _passc_body(z2_ref, st2_ref, gb2_ref, wv2t_ref, nf_ref):
    z2s = [z2_ref[0, d] for d in range(3)]
    factor = _bn_factor(z2s, st2_ref, gb2_ref)
    xs = [z * factor for z in z2s]
    fs = _vn_block(xs, wv2t_ref[...])
    C2 = fs[0].shape[-1]
    nrm = fs[0] * fs[0] + fs[1] * fs[1] + fs[2] * fs[2]
    ng = nrm.reshape(TILE // S, S, C2)
    m = jnp.max(ng, axis=1, keepdims=True)
    si = jax.lax.broadcasted_iota(jnp.int32, (TILE // S, S, C2), 1)
    cand = jnp.where(ng == m, si, S)
    smin = jnp.min(cand, axis=1, keepdims=True)
    oh = (si == smin).astype(jnp.float32)
    for d in range(3):
        nf_ref[0, d] = jnp.sum(fs[d].reshape(TILE // S, S, C2) * oh, axis=1)


def _run_passc(z2, st2, gb2, wv2t):
    C2 = wv2t.shape[0]
    NG = TILE // S
    return pl.pallas_call(
        _passc_body,
        grid=(GRID,),
        in_specs=[pl.BlockSpec((1, 3, TILE, C2), lambda t: (t, 0, 0, 0)),
                  pl.BlockSpec((8, C2), lambda t: (0, 0)),
                  pl.BlockSpec((2, C2), lambda t: (0, 0)),
                  pl.BlockSpec((C2, C2), lambda t: (0, 0))],
        out_specs=pl.BlockSpec((1, 3, NG, C2), lambda t: (t, 0, 0, 0)),
        out_shape=jax.ShapeDtypeStruct((GRID, 3, NG, C2), jnp.float32),
    )(z2, st2, gb2, wv2t)


def kernel(xyz, points, prev_feat, W1, g1, b1, Wv1, W2, g2, b2, Wv2):
    del points
    xyz = xyz.astype(jnp.float32)
    xyz_rows = jnp.transpose(xyz, (0, 2, 1))

    fps3, new_q = _run_fps(xyz, xyz_rows)
    fps_idx = fps3.reshape(B, G)
    new_xyz = jnp.transpose(new_q, (0, 2, 1))

    idx = _run_knn(new_q, xyz)

    pf_t = jnp.transpose(prev_feat, (0, 3, 2, 1))
    tbl = jnp.concatenate(
        [xyz[:, :, :, None], pf_t,
         jnp.zeros((B, 3, N, 15), jnp.float32)], axis=3)
    w1t = jnp.pad(W1.T, ((0, 15), (0, 0)))

    idxt = idx.reshape(GRID, TILE, 1)
    qt = jnp.pad(jnp.repeat(new_q, S, axis=1).reshape(GRID, TILE, 3),
                 ((0, 0), (0, 0), (0, 1)))

    z1 = _run_passa(tbl, idxt, qt, w1t)

    def _stats(z, c):
        zm = (z.reshape(B, NT, 3, TILE, c).transpose(0, 4, 1, 3, 2)
              .reshape(B, c, T, 3))
        nrm = jnp.sqrt(jnp.sum(zm ** 2, axis=-1) + 1e-8)
        return jnp.pad(jnp.stack([jnp.mean(nrm, axis=(0, 2)),
                                  jnp.var(nrm, axis=(0, 2))]), ((0, 6), (0, 0)))

    st1 = _stats(z1, 64)
    gb1 = jnp.stack([g1, b1])
    z2 = _run_passb(z1, st1, gb1, Wv1.T, W2.T)
    st2 = _stats(z2, 128)
    gb2 = jnp.stack([g2, b2])
    nf = _run_passc(z2, st2, gb2, Wv2.T)

    new_feat = nf.reshape(B, NT, 3, TILE // S, 128)
    new_feat = new_feat.transpose(0, 4, 1, 3, 2).reshape(B, 128, G, 3)
    return new_xyz, new_feat, fps_idx
